# Initial kernel scaffold; baseline (speedup 1.0000x reference)
import jax
import jax.numpy as jnp
from jax.experimental import pallas as pl


def kernel(x, table, bias):
    raise NotImplementedError("write your pallas kernel here")


# speedup vs baseline: 1.2520x; 1.2520x over previous
import functools

import jax
import jax.numpy as jnp
from jax import lax
from jax.experimental import pallas as pl
from jax.experimental.pallas import tpu as pltpu
from jax.experimental.pallas import tpu_sc as plsc

B = 16384
F = 26
FIELD = 40000
NW = 32
RPW = B // NW
EPW = RPW * F
L = 16


def _sc_kernel(x_hbm, table_hbm, bias_hbm, out_hbm,
               xs_v, idx_v, vals_v, bias_v, out_v, sem):
    wid = lax.axis_index("s") * 2 + lax.axis_index("c")
    ebase = wid * EPW

    pltpu.sync_copy(x_hbm.at[pl.ds(ebase, EPW)], xs_v)
    pltpu.sync_copy(bias_hbm, bias_v)

    iota = lax.iota(jnp.int32, L)

    def idx_body(c, carry):
        pos = c * L + iota
        f = lax.rem(pos, F)
        idx_v[pl.ds(c * L, L)] = xs_v[pl.ds(c * L, L)] + f * FIELD
        return carry
    lax.fori_loop(0, EPW // L, idx_body, 0)

    pltpu.async_copy(table_hbm.at[idx_v], vals_v, sem).wait()

    bias_vec = bias_v[...]

    def red_body(j, carry):
        lanes = (j * L + iota) * F
        acc = bias_vec
        for f in range(F):
            acc = acc + plsc.load_gather(vals_v, [lanes + f])
        out_v[pl.ds(j * L, L)] = acc
        return carry
    lax.fori_loop(0, RPW // L, red_body, 0)

    pltpu.sync_copy(out_v, out_hbm.at[pl.ds(wid * RPW, RPW)])


@jax.jit
def _features_linear(x_flat, table_flat, bias16):
    mesh = plsc.VectorSubcoreMesh(core_axis_name="c", subcore_axis_name="s")
    run = functools.partial(
        pl.kernel,
        mesh=mesh,
        compiler_params=pltpu.CompilerParams(needs_layout_passes=False),
        out_type=jax.ShapeDtypeStruct((B,), jnp.float32),
        scratch_types=[
            pltpu.VMEM((EPW,), jnp.int32),
            pltpu.VMEM((EPW,), jnp.int32),
            pltpu.VMEM((EPW,), jnp.float32),
            pltpu.VMEM((L,), jnp.float32),
            pltpu.VMEM((RPW,), jnp.float32),
            pltpu.SemaphoreType.DMA,
        ],
    )(_sc_kernel)
    return run(x_flat, table_flat, bias16)


def kernel(x, table, bias):
    x_flat = x.reshape(-1).astype(jnp.int32)
    table_flat = table.reshape(-1)
    bias16 = jnp.broadcast_to(bias.astype(jnp.float32), (L,))
    out = _features_linear(x_flat, table_flat, bias16)
    return out.reshape(B, 1)


# speedup vs baseline: 1.4885x; 1.1888x over previous
import functools

import jax
import jax.numpy as jnp
from jax import lax
from jax.experimental import pallas as pl
from jax.experimental.pallas import tpu as pltpu
from jax.experimental.pallas import tpu_sc as plsc

B = 16384
F = 26
FIELD = 40000
NW = 32
RPW = B // NW
EPW = RPW * F
L = 16


def _sc_kernel(xt_hbm, table_hbm, bias_hbm, out_hbm,
               idx_v, vals_v, bias_v, out_v, sem, gsem):
    wid = lax.axis_index("s") * 2 + lax.axis_index("c")
    base = wid * RPW

    for f in range(F):
        pltpu.async_copy(xt_hbm.at[f, pl.ds(base, RPW)],
                         idx_v.at[pl.ds(f * RPW, RPW)], sem)
    pltpu.sync_copy(bias_hbm, bias_v)
    for f in range(F):
        pltpu.make_async_copy(xt_hbm.at[f, pl.ds(base, RPW)],
                              idx_v.at[pl.ds(f * RPW, RPW)], sem).wait()

    iota = lax.iota(jnp.int32, L)

    def idx_body(c, carry):
        f = c >> 5
        p = c * L
        idx_v[pl.ds(p, L)] = idx_v[pl.ds(p, L)] + f * FIELD
        return carry
    lax.fori_loop(0, EPW // L, idx_body, 0)

    pltpu.async_copy(table_hbm.at[idx_v], vals_v, gsem).wait()

    bias_vec = bias_v[...]

    def red_body(j, carry):
        p = j * L
        acc = bias_vec
        for f in range(F):
            acc = acc + vals_v[pl.ds(f * RPW + p, L)]
        out_v[pl.ds(p, L)] = acc
        return carry
    lax.fori_loop(0, RPW // L, red_body, 0)

    pltpu.sync_copy(out_v, out_hbm.at[pl.ds(base, RPW)])


@jax.jit
def _features_linear(xt, table_flat, bias16):
    mesh = plsc.VectorSubcoreMesh(core_axis_name="c", subcore_axis_name="s")
    run = functools.partial(
        pl.kernel,
        mesh=mesh,
        compiler_params=pltpu.CompilerParams(needs_layout_passes=False),
        out_type=jax.ShapeDtypeStruct((B,), jnp.float32),
        scratch_types=[
            pltpu.VMEM((EPW,), jnp.int32),
            pltpu.VMEM((EPW,), jnp.float32),
            pltpu.VMEM((L,), jnp.float32),
            pltpu.VMEM((RPW,), jnp.float32),
            pltpu.SemaphoreType.DMA,
            pltpu.SemaphoreType.DMA,
        ],
    )(_sc_kernel)
    return run(xt, table_flat, bias16)


def kernel(x, table, bias):
    xt = x.T.astype(jnp.int32)
    table_flat = (table + jnp.float32(0.0)).reshape(-1)
    bias16 = jnp.broadcast_to(bias.astype(jnp.float32), (L,))
    out = _features_linear(xt, table_flat, bias16)
    return out.reshape(B, 1)


# speedup vs baseline: 3.8656x; 2.5970x over previous
import functools

import jax
import jax.numpy as jnp
from jax import lax
from jax.experimental import pallas as pl
from jax.experimental.pallas import tpu as pltpu
from jax.experimental.pallas import tpu_sc as plsc

B = 16384
F = 26
FIELD = 40000
TOTAL = F * FIELD
NW = 32
RPW = B // NW
EPW = RPW * F
L = 16
CH = 65024
CHL = TOTAL - 15 * CH


def _sc_kernel(xt_hbm, table_hbm, bias_hbm, out_hbm,
               idx_v, vals_v, bias_v, out_v, stab,
               xsem, fsem, gsem):
    cid = lax.axis_index("c")
    sid = lax.axis_index("s")
    wid = sid * 2 + cid
    base = wid * RPW

    for f in range(F):
        pltpu.async_copy(xt_hbm.at[f, pl.ds(base, RPW)],
                         idx_v.at[pl.ds(f * RPW, RPW)], xsem)

    @pl.when(sid < 15)
    def _():
        pltpu.async_copy(table_hbm.at[0, pl.ds(sid * CH, CH)],
                         stab.at[pl.ds(sid * CH, CH)], fsem)

    @pl.when(sid == 15)
    def _():
        pltpu.async_copy(table_hbm.at[0, pl.ds(15 * CH, CHL)],
                         stab.at[pl.ds(15 * CH, CHL)], fsem)

    pltpu.sync_copy(bias_hbm, bias_v)

    for f in range(F):
        pltpu.make_async_copy(xt_hbm.at[f, pl.ds(base, RPW)],
                              idx_v.at[pl.ds(f * RPW, RPW)], xsem).wait()

    iota = lax.iota(jnp.int32, L)

    def idx_body(c, carry):
        f = c >> 5
        p = c * L
        idx_v[pl.ds(p, L)] = idx_v[pl.ds(p, L)] + f * FIELD
        return carry
    lax.fori_loop(0, EPW // L, idx_body, 0)

    @pl.when(sid < 15)
    def _():
        pltpu.make_async_copy(table_hbm.at[0, pl.ds(sid * CH, CH)],
                              stab.at[pl.ds(sid * CH, CH)], fsem).wait()

    @pl.when(sid == 15)
    def _():
        pltpu.make_async_copy(table_hbm.at[0, pl.ds(15 * CH, CHL)],
                              stab.at[pl.ds(15 * CH, CHL)], fsem).wait()

    plsc.subcore_barrier()

    pltpu.async_copy(stab.at[idx_v], vals_v, gsem).wait()

    bias_vec = bias_v[...]

    def red_body(j, carry):
        p = j * L
        acc = bias_vec
        for f in range(F):
            acc = acc + vals_v[pl.ds(f * RPW + p, L)]
        out_v[pl.ds(p, L)] = acc
        return carry
    lax.fori_loop(0, RPW // L, red_body, 0)

    pltpu.sync_copy(out_v, out_hbm.at[pl.ds(base, RPW)])


@jax.jit
def _features_linear(xt, table2, bias16):
    mesh = plsc.VectorSubcoreMesh(core_axis_name="c", subcore_axis_name="s")
    run = functools.partial(
        pl.kernel,
        mesh=mesh,
        compiler_params=pltpu.CompilerParams(needs_layout_passes=False),
        out_type=jax.ShapeDtypeStruct((B,), jnp.float32),
        scratch_types=[
            pltpu.VMEM((EPW,), jnp.int32),
            pltpu.VMEM((EPW,), jnp.float32),
            pltpu.VMEM((L,), jnp.float32),
            pltpu.VMEM((RPW,), jnp.float32),
            pltpu.VMEM_SHARED((TOTAL,), jnp.float32),
            pltpu.SemaphoreType.DMA,
            pltpu.SemaphoreType.DMA,
            pltpu.SemaphoreType.DMA,
        ],
    )(_sc_kernel)
    return run(xt, table2, bias16)


def kernel(x, table, bias):
    xt = x.T.astype(jnp.int32)
    bias16 = jnp.broadcast_to(bias.astype(jnp.float32), (L,))
    out = _features_linear(xt, table.T, bias16)
    return out.reshape(B, 1)


# speedup vs baseline: 3.9417x; 1.0197x over previous
import functools

import jax
import jax.numpy as jnp
from jax import lax
from jax.experimental import pallas as pl
from jax.experimental.pallas import tpu as pltpu
from jax.experimental.pallas import tpu_sc as plsc

B = 16384
F = 26
FIELD = 40000
TOTAL = F * FIELD
NW = 32
RPW = B // NW
EPW = RPW * F
L = 16
CH = 65024
CHL = TOTAL - 15 * CH


def _sc_kernel(xt_hbm, table_hbm, bias_hbm, out_hbm,
               idx_v, vals_v, bias_v, out_v, stab,
               xsem, fsem, gsem):
    cid = lax.axis_index("c")
    sid = lax.axis_index("s")
    wid = sid * 2 + cid
    base = wid * RPW

    for f in range(F):
        pltpu.async_copy(xt_hbm.at[f, pl.ds(base, RPW)],
                         idx_v.at[pl.ds(f * RPW, RPW)], xsem)

    @pl.when(sid < 15)
    def _():
        pltpu.async_copy(table_hbm.at[0, pl.ds(sid * CH, CH)],
                         stab.at[pl.ds(sid * CH, CH)], fsem)

    @pl.when(sid == 15)
    def _():
        pltpu.async_copy(table_hbm.at[0, pl.ds(15 * CH, CHL)],
                         stab.at[pl.ds(15 * CH, CHL)], fsem)

    pltpu.sync_copy(bias_hbm, bias_v)

    for f in range(F):
        pltpu.make_async_copy(xt_hbm.at[f, pl.ds(base, RPW)],
                              idx_v.at[pl.ds(f * RPW, RPW)], xsem).wait()

    with jax.named_scope("idx_add"):
        def idx_body(c, carry):
            f = c >> 2
            off = f * FIELD
            p = c * 128
            for k in range(8):
                q = p + k * L
                idx_v[pl.ds(q, L)] = idx_v[pl.ds(q, L)] + off
            return carry
        lax.fori_loop(0, EPW // 128, idx_body, 0)

    with jax.named_scope("fill_wait"):
        @pl.when(sid < 15)
        def _():
            pltpu.make_async_copy(table_hbm.at[0, pl.ds(sid * CH, CH)],
                                  stab.at[pl.ds(sid * CH, CH)], fsem).wait()

        @pl.when(sid == 15)
        def _():
            pltpu.make_async_copy(table_hbm.at[0, pl.ds(15 * CH, CHL)],
                                  stab.at[pl.ds(15 * CH, CHL)], fsem).wait()

        plsc.subcore_barrier()

    with jax.named_scope("gather"):
        pltpu.async_copy(stab.at[idx_v], vals_v, gsem).wait()

    bias_vec = bias_v[...]

    with jax.named_scope("reduce"):
        def red_body(j, carry):
            p = j * L
            acc = bias_vec
            for f in range(F):
                acc = acc + vals_v[pl.ds(f * RPW + p, L)]
            out_v[pl.ds(p, L)] = acc
            return carry
        lax.fori_loop(0, RPW // L, red_body, 0)

    pltpu.sync_copy(out_v, out_hbm.at[pl.ds(base, RPW)])


@jax.jit
def _features_linear(xt, table2, bias16):
    mesh = plsc.VectorSubcoreMesh(core_axis_name="c", subcore_axis_name="s")
    run = functools.partial(
        pl.kernel,
        mesh=mesh,
        compiler_params=pltpu.CompilerParams(needs_layout_passes=False),
        out_type=jax.ShapeDtypeStruct((B,), jnp.float32),
        scratch_types=[
            pltpu.VMEM((EPW,), jnp.int32),
            pltpu.VMEM((EPW,), jnp.float32),
            pltpu.VMEM((L,), jnp.float32),
            pltpu.VMEM((RPW,), jnp.float32),
            pltpu.VMEM_SHARED((TOTAL,), jnp.float32),
            pltpu.SemaphoreType.DMA,
            pltpu.SemaphoreType.DMA,
            pltpu.SemaphoreType.DMA,
        ],
    )(_sc_kernel)
    return run(xt, table2, bias16)


def kernel(x, table, bias):
    xt = x.T.astype(jnp.int32)
    bias16 = jnp.broadcast_to(bias.astype(jnp.float32), (L,))
    out = _features_linear(xt, table.T, bias16)
    return out.reshape(B, 1)
